# baseline (speedup 1.0000x reference)
import functools

import jax
import jax.numpy as jnp
from jax import lax
from jax.experimental import pallas as pl
from jax.experimental.pallas import tpu as pltpu
from jax.experimental.pallas import tpu_sc as plsc

_info = plsc.get_sparse_core_info()
_NC, _NS, _L = _info.num_cores, _info.num_subcores, _info.num_lanes


def _make_sc_counts(B, C):
    per_sub = B // _NS
    n_cnt_chunks = per_sub // _L

    mesh = plsc.VectorSubcoreMesh(core_axis_name="c", subcore_axis_name="s")

    @functools.partial(
        pl.kernel,
        mesh=mesh,
        out_type=jax.ShapeDtypeStruct((_L,), jnp.float32),
        scratch_types=[
            pltpu.VMEM((per_sub,), jnp.int32),
            pltpu.VMEM((_L,), jnp.float32),
            pltpu.VMEM((_NS * _L,), jnp.float32),
            pltpu.VMEM_SHARED((_NS * _L,), jnp.float32),
        ],
    )
    def sc_counts(ids_hbm, cnt_hbm, ids_v, cnt_v, all_v, shared):
        cid = lax.axis_index("c")
        sid = lax.axis_index("s")
        pltpu.sync_copy(ids_hbm.at[pl.ds(sid * per_sub, per_sub)], ids_v)

        zero = jnp.zeros((_L,), jnp.int32)
        lane = lax.iota(jnp.int32, _L)

        def lane_sum(x):
            for k in (1, 2, 4, 8):
                x = x + x.at[lane ^ k].get(mode="promise_in_bounds")
            return x

        cntvec = jnp.zeros((_L,), jnp.float32)
        if C == 4:
            def cbody(j, accs):
                a01, a10, a11 = accs
                v = ids_v[pl.ds(j * _L, _L)]
                b0 = v & 1
                b1 = v >> 1
                return (a01 + b0, a10 + b1, a11 + b0 * b1)

            a01, a10, a11 = lax.fori_loop(0, n_cnt_chunks, cbody, (zero,) * 3)
            s1 = lane_sum(a01)
            s2 = lane_sum(a10)
            s3 = lane_sum(a11)
            n = jnp.full((_L,), per_sub, jnp.int32)
            bins = (n - s1 - s2 + s3, s1 - s3, s2 - s3, s3)
            for c in range(4):
                cntvec = jnp.where(lane == c, bins[c].astype(jnp.float32), cntvec)
        else:
            def cbody(j, accs):
                v = ids_v[pl.ds(j * _L, _L)]
                one = jnp.ones((_L,), jnp.int32)
                return tuple(a + jnp.where(v == c, one, zero) for c, a in enumerate(accs))

            accs = lax.fori_loop(0, n_cnt_chunks, cbody, (zero,) * C)
            for c in range(C):
                cnt_c = lane_sum(accs[c]).astype(jnp.float32)
                cntvec = jnp.where(lane == c, cnt_c, cntvec)

        cnt_v[...] = cntvec
        pltpu.sync_copy(cnt_v, shared.at[pl.ds(sid * _L, _L)])
        plsc.subcore_barrier()
        pltpu.sync_copy(shared, all_v)

        def tbody(j, tot):
            return tot + all_v[pl.ds(j * _L, _L)]

        totals = lax.fori_loop(0, _NS, tbody, jnp.zeros((_L,), jnp.float32))

        @pl.when(jnp.logical_and(cid == 0, sid == 0))
        def _write():
            cnt_v[...] = totals
            pltpu.sync_copy(cnt_v, cnt_hbm)

    return sc_counts


def _tc_apply(B, counts_ref, tr_ref, ids_col_ref, x_ref, out_ref, sf_ref):
    i = pl.program_id(0)
    C = tr_ref.shape[0]

    @pl.when(i == 0)
    def _factors():
        for c in range(C):
            cur_ratio = counts_ref[c] / float(B)
            sf_ref[c] = tr_ref[c] / cur_ratio

    ids_col = ids_col_ref[...]
    s = jnp.full(ids_col.shape, sf_ref[C - 1], dtype=jnp.float32)
    for c in range(C - 2, -1, -1):
        s = jnp.where(ids_col == c, sf_ref[c], s)
    x = x_ref[...]
    xs = x * s
    out_ref[...] = xs + (x - xs)


def kernel(input, target_ratios, class_ids):
    B, D = input.shape
    C = target_ratios.shape[0]
    ids = class_ids.astype(jnp.int32)

    counts = _make_sc_counts(B, C)(ids)
    ids_col = ids.reshape(B, 1)

    R = 8192
    grid = (B // R,)
    return pl.pallas_call(
        functools.partial(_tc_apply, B),
        grid=grid,
        in_specs=[
            pl.BlockSpec(memory_space=pltpu.SMEM),
            pl.BlockSpec(memory_space=pltpu.SMEM),
            pl.BlockSpec((R, 1), lambda i: (i, 0)),
            pl.BlockSpec((R, D), lambda i: (i, 0)),
        ],
        out_specs=pl.BlockSpec((R, D), lambda i: (i, 0)),
        out_shape=jax.ShapeDtypeStruct((B, D), jnp.float32),
        scratch_shapes=[pltpu.SMEM((C,), jnp.float32)],
    )(counts, target_ratios, ids_col, input)


# speedup vs baseline: 1.0497x; 1.0497x over previous
import functools

import jax
import jax.numpy as jnp
from jax import lax
from jax.experimental import pallas as pl
from jax.experimental.pallas import tpu as pltpu
from jax.experimental.pallas import tpu_sc as plsc

_info = plsc.get_sparse_core_info()
_NC, _NS, _L = _info.num_cores, _info.num_subcores, _info.num_lanes


def _make_sc_counts(B, C):
    per_sub = B // _NS
    n_cnt_chunks = per_sub // _L

    mesh = plsc.VectorSubcoreMesh(
        core_axis_name="c", subcore_axis_name="s", num_cores=1
    )

    @functools.partial(
        pl.kernel,
        mesh=mesh,
        out_type=jax.ShapeDtypeStruct((_L,), jnp.float32),
        scratch_types=[
            pltpu.VMEM((per_sub,), jnp.int32),
            pltpu.VMEM((_L,), jnp.float32),
            pltpu.VMEM((_NS * _L,), jnp.float32),
            pltpu.VMEM_SHARED((_NS * _L,), jnp.float32),
        ],
    )
    def sc_counts(ids_hbm, cnt_hbm, ids_v, cnt_v, all_v, shared):
        cid = lax.axis_index("c")
        sid = lax.axis_index("s")
        pltpu.sync_copy(ids_hbm.at[pl.ds(sid * per_sub, per_sub)], ids_v)

        zero = jnp.zeros((_L,), jnp.int32)
        lane = lax.iota(jnp.int32, _L)

        def lane_sum(x):
            for k in (1, 2, 4, 8):
                x = x + x.at[lane ^ k].get(mode="promise_in_bounds")
            return x

        cntvec = jnp.zeros((_L,), jnp.float32)
        if C == 4:
            def cbody(j, accs):
                a01, a10, a11 = accs
                v = ids_v[pl.ds(j * _L, _L)]
                b0 = v & 1
                b1 = v >> 1
                return (a01 + b0, a10 + b1, a11 + b0 * b1)

            a01, a10, a11 = lax.fori_loop(0, n_cnt_chunks, cbody, (zero,) * 3)
            s1 = lane_sum(a01)
            s2 = lane_sum(a10)
            s3 = lane_sum(a11)
            n = jnp.full((_L,), per_sub, jnp.int32)
            bins = (n - s1 - s2 + s3, s1 - s3, s2 - s3, s3)
            for c in range(4):
                cntvec = jnp.where(lane == c, bins[c].astype(jnp.float32), cntvec)
        else:
            def cbody(j, accs):
                v = ids_v[pl.ds(j * _L, _L)]
                one = jnp.ones((_L,), jnp.int32)
                return tuple(a + jnp.where(v == c, one, zero) for c, a in enumerate(accs))

            accs = lax.fori_loop(0, n_cnt_chunks, cbody, (zero,) * C)
            for c in range(C):
                cnt_c = lane_sum(accs[c]).astype(jnp.float32)
                cntvec = jnp.where(lane == c, cnt_c, cntvec)

        cnt_v[...] = cntvec
        pltpu.sync_copy(cnt_v, shared.at[pl.ds(sid * _L, _L)])
        plsc.subcore_barrier()
        pltpu.sync_copy(shared, all_v)

        def tbody(j, tot):
            return tot + all_v[pl.ds(j * _L, _L)]

        totals = lax.fori_loop(0, _NS, tbody, jnp.zeros((_L,), jnp.float32))

        @pl.when(jnp.logical_and(cid == 0, sid == 0))
        def _write():
            cnt_v[...] = totals
            pltpu.sync_copy(cnt_v, cnt_hbm)

    return sc_counts


def _tc_apply(B, counts_ref, tr_ref, ids_col_ref, x_ref, out_ref, sf_ref):
    i = pl.program_id(0)
    C = tr_ref.shape[0]

    @pl.when(i == 0)
    def _factors():
        for c in range(C):
            cur_ratio = counts_ref[c] / float(B)
            sf_ref[c] = tr_ref[c] / cur_ratio

    ids_col = ids_col_ref[...]
    s = jnp.full(ids_col.shape, sf_ref[C - 1], dtype=jnp.float32)
    for c in range(C - 2, -1, -1):
        s = jnp.where(ids_col == c, sf_ref[c], s)
    x = x_ref[...]
    xs = x * s
    out_ref[...] = xs + (x - xs)


def kernel(input, target_ratios, class_ids):
    B, D = input.shape
    C = target_ratios.shape[0]
    ids = class_ids.astype(jnp.int32)

    counts = _make_sc_counts(B, C)(ids)
    ids_col = ids.reshape(B, 1)

    R = 8192
    grid = (B // R,)
    return pl.pallas_call(
        functools.partial(_tc_apply, B),
        grid=grid,
        in_specs=[
            pl.BlockSpec(memory_space=pltpu.SMEM),
            pl.BlockSpec(memory_space=pltpu.SMEM),
            pl.BlockSpec((R, 1), lambda i: (i, 0)),
            pl.BlockSpec((R, D), lambda i: (i, 0)),
        ],
        out_specs=pl.BlockSpec((R, D), lambda i: (i, 0)),
        out_shape=jax.ShapeDtypeStruct((B, D), jnp.float32),
        scratch_shapes=[pltpu.SMEM((C,), jnp.float32)],
    )(counts, target_ratios, ids_col, input)


# speedup vs baseline: 1.0520x; 1.0022x over previous
import functools

import jax
import jax.numpy as jnp
from jax import lax
from jax.experimental import pallas as pl
from jax.experimental.pallas import tpu as pltpu
from jax.experimental.pallas import tpu_sc as plsc

_info = plsc.get_sparse_core_info()
_NS, _L = _info.num_subcores, _info.num_lanes


def _make_sc_counts(B, C):
    per_sub = B // _NS
    n_cnt_chunks = per_sub // _L

    mesh = plsc.VectorSubcoreMesh(
        core_axis_name="c", subcore_axis_name="s", num_cores=1
    )

    @functools.partial(
        pl.kernel,
        mesh=mesh,
        out_type=jax.ShapeDtypeStruct((_L,), jnp.float32),
        scratch_types=[
            pltpu.VMEM((per_sub,), jnp.int32),
            pltpu.VMEM((_L,), jnp.float32),
            pltpu.VMEM((_NS * _L,), jnp.float32),
            pltpu.VMEM_SHARED((_NS * _L,), jnp.float32),
        ],
    )
    def sc_counts(ids_hbm, cnt_hbm, ids_v, cnt_v, all_v, shared):
        cid = lax.axis_index("c")
        sid = lax.axis_index("s")
        pltpu.sync_copy(ids_hbm.at[pl.ds(sid * per_sub, per_sub)], ids_v)

        zero = jnp.zeros((_L,), jnp.int32)
        lane = lax.iota(jnp.int32, _L)

        def lane_sum(x):
            for k in (1, 2, 4, 8):
                x = x + x.at[lane ^ k].get(mode="promise_in_bounds")
            return x

        cntvec = jnp.zeros((_L,), jnp.float32)
        if C == 4:
            def cbody(j, accs):
                a01, a10, a11 = accs
                v = ids_v[pl.ds(j * _L, _L)]
                b0 = v & 1
                b1 = v >> 1
                return (a01 + b0, a10 + b1, a11 + b0 * b1)

            a01, a10, a11 = lax.fori_loop(0, n_cnt_chunks, cbody, (zero,) * 3)
            s1 = lane_sum(a01)
            s2 = lane_sum(a10)
            s3 = lane_sum(a11)
            n = jnp.full((_L,), per_sub, jnp.int32)
            bins = (n - s1 - s2 + s3, s1 - s3, s2 - s3, s3)
            for c in range(4):
                cntvec = jnp.where(lane == c, bins[c].astype(jnp.float32), cntvec)
        else:
            def cbody(j, accs):
                v = ids_v[pl.ds(j * _L, _L)]
                one = jnp.ones((_L,), jnp.int32)
                return tuple(a + jnp.where(v == c, one, zero) for c, a in enumerate(accs))

            accs = lax.fori_loop(0, n_cnt_chunks, cbody, (zero,) * C)
            for c in range(C):
                cnt_c = lane_sum(accs[c]).astype(jnp.float32)
                cntvec = jnp.where(lane == c, cnt_c, cntvec)

        cnt_v[...] = cntvec
        pltpu.sync_copy(cnt_v, shared.at[pl.ds(sid * _L, _L)])
        plsc.subcore_barrier()
        pltpu.sync_copy(shared, all_v)

        def tbody(j, tot):
            return tot + all_v[pl.ds(j * _L, _L)]

        totals = lax.fori_loop(0, _NS, tbody, jnp.zeros((_L,), jnp.float32))

        @pl.when(jnp.logical_and(cid == 0, sid == 0))
        def _write():
            cnt_v[...] = totals
            pltpu.sync_copy(cnt_v, cnt_hbm)

    return sc_counts


def _tc_apply(B, counts_ref, tr_ref, ids_col_ref, x_ref, out_ref, sf_ref):
    i = pl.program_id(0)
    C = tr_ref.shape[0]

    @pl.when(i == 0)
    def _factors():
        for c in range(C):
            cur_ratio = counts_ref[c] / float(B)
            sf_ref[c] = tr_ref[c] / cur_ratio

    ids_col = ids_col_ref[...]
    s = jnp.full(ids_col.shape, sf_ref[C - 1], dtype=jnp.float32)
    for c in range(C - 2, -1, -1):
        s = jnp.where(ids_col == c, sf_ref[c], s)
    x = x_ref[...]
    xs = x * s
    out_ref[...] = xs + (x - xs)


def kernel(input, target_ratios, class_ids):
    B, D = input.shape
    C = target_ratios.shape[0]
    ids = class_ids.astype(jnp.int32)

    counts = _make_sc_counts(B, C)(ids)
    ids_col = ids.reshape(B, 1)

    R = 8192
    grid = (B // R,)
    return pl.pallas_call(
        functools.partial(_tc_apply, B),
        grid=grid,
        in_specs=[
            pl.BlockSpec(memory_space=pltpu.SMEM),
            pl.BlockSpec(memory_space=pltpu.SMEM),
            pl.BlockSpec((R, 1), lambda i: (i, 0)),
            pl.BlockSpec((R, D), lambda i: (i, 0)),
        ],
        out_specs=pl.BlockSpec((R, D), lambda i: (i, 0)),
        out_shape=jax.ShapeDtypeStruct((B, D), jnp.float32),
        scratch_shapes=[pltpu.SMEM((C,), jnp.float32)],
    )(counts, target_ratios, ids_col, input)
